# baseline (speedup 1.0000x reference)
import jax
import jax.numpy as jnp
from jax.experimental import pallas as pl
from jax.experimental.pallas import tpu as pltpu

B, N, V, D = 4096, 20, 1000, 64
BB = 256


def _cbow_kernel(x_ref, we_ref, wo_ref, b_ref, out_ref):
    h = jax.lax.dot(x_ref[...], we_ref[...],
                    preferred_element_type=jnp.float32)
    logits = jax.lax.dot(h, wo_ref[...],
                         preferred_element_type=jnp.float32)
    logits = logits + b_ref[...]
    m = jnp.max(logits, axis=-1, keepdims=True)
    e = jnp.exp(logits - m)
    out_ref[...] = e / jnp.sum(e, axis=-1, keepdims=True)


@jax.jit
def kernel(inputs, W_emb, W_out, b_out):
    x2 = inputs.reshape(B, N * V)
    we_tiled = jnp.tile(W_emb * (1.0 / N), (N, 1))
    b2 = b_out.reshape(1, V)
    grid = (B // BB,)
    return pl.pallas_call(
        _cbow_kernel,
        grid=grid,
        in_specs=[
            pl.BlockSpec((BB, N * V), lambda i: (i, 0)),
            pl.BlockSpec((N * V, D), lambda i: (0, 0)),
            pl.BlockSpec((D, V), lambda i: (0, 0)),
            pl.BlockSpec((1, V), lambda i: (0, 0)),
        ],
        out_specs=pl.BlockSpec((BB, V), lambda i: (i, 0)),
        out_shape=jax.ShapeDtypeStruct((B, V), jnp.float32),
        compiler_params=pltpu.CompilerParams(
            dimension_semantics=("arbitrary",),
        ),
    )(x2, we_tiled, W_out, b2)
